# baseline (speedup 1.0000x reference)
import functools

import jax
import jax.numpy as jnp
from jax import lax
from jax.experimental import pallas as pl
from jax.experimental.pallas import tpu as pltpu
from jax.experimental.pallas import tpu_sc as plsc

VOCAB = 1000000
DIM = 32
BATCH = 4096
HIST = 50

NUM_CORES = 2
NUM_SUBCORES = 16
NUM_WORKERS = NUM_CORES * NUM_SUBCORES

TOTAL = BATCH * HIST
PER_WORKER = TOTAL // NUM_WORKERS
CHUNK = 128
NCHUNK = PER_WORKER // CHUNK

_mesh = plsc.VectorSubcoreMesh(core_axis_name="c", subcore_axis_name="s")


@functools.partial(
    pl.kernel,
    mesh=_mesh,
    out_type=jax.ShapeDtypeStruct((TOTAL, DIM), jnp.float32),
    scratch_types=[
        pltpu.VMEM((NCHUNK, CHUNK), jnp.int32),
        pltpu.VMEM((CHUNK, DIM), jnp.float32),
        pltpu.SemaphoreType.DMA,
    ],
    compiler_params=pltpu.CompilerParams(use_tc_tiling_on_sc=False),
)
def _gather_kernel(idx_hbm, table_hbm, out_hbm, idx_v, rows_v, gsem):
    wid = lax.axis_index("s") * NUM_CORES + lax.axis_index("c")
    base = wid * PER_WORKER
    pltpu.sync_copy(idx_hbm.at[wid], idx_v)

    def body(j, carry):
        idx_row = idx_v.at[j]
        pltpu.async_copy(table_hbm.at[idx_row], rows_v, gsem).wait()
        pltpu.sync_copy(rows_v, out_hbm.at[pl.ds(base + j * CHUNK, CHUNK)])
        return carry

    lax.fori_loop(0, NCHUNK, body, 0, unroll=False)


def kernel(indices, table):
    idx2d = indices.astype(jnp.int32).reshape(NUM_WORKERS, NCHUNK, CHUNK)
    out = _gather_kernel(idx2d, table)
    return out.reshape(BATCH, HIST, DIM)


# speedup vs baseline: 1.0465x; 1.0465x over previous
import functools

import jax
import jax.numpy as jnp
from jax import lax
from jax.experimental import pallas as pl
from jax.experimental.pallas import tpu as pltpu
from jax.experimental.pallas import tpu_sc as plsc

VOCAB = 1000000
DIM = 32
BATCH = 4096
HIST = 50

NUM_CORES = 2
NUM_SUBCORES = 16
NUM_WORKERS = NUM_CORES * NUM_SUBCORES

TOTAL = BATCH * HIST
PER_WORKER = TOTAL // NUM_WORKERS
CHUNK = 128
NCHUNK = PER_WORKER // CHUNK
NBUF = 8
DEPTH = 6

_mesh = plsc.VectorSubcoreMesh(core_axis_name="c", subcore_axis_name="s")


@functools.partial(
    pl.kernel,
    mesh=_mesh,
    out_type=jax.ShapeDtypeStruct((TOTAL, DIM), jnp.float32),
    scratch_types=[
        pltpu.VMEM((NCHUNK, CHUNK), jnp.int32),
        pltpu.VMEM((NBUF, CHUNK, DIM), jnp.float32),
        pltpu.SemaphoreType.DMA((NBUF,)),
        pltpu.SemaphoreType.DMA((NBUF,)),
    ],
    compiler_params=pltpu.CompilerParams(use_tc_tiling_on_sc=False),
)
def _gather_kernel(idx_hbm, table_hbm, out_hbm, idx_v, rows_v, gsem, ssem):
    wid = lax.axis_index("s") * NUM_CORES + lax.axis_index("c")
    base = wid * PER_WORKER
    pltpu.sync_copy(idx_hbm.at[wid], idx_v)

    def issue_gather(n, b):
        pltpu.async_copy(table_hbm.at[idx_v.at[n]], rows_v.at[b], gsem.at[b])

    def wait_gather(b):
        pltpu.make_async_copy(
            out_hbm.at[pl.ds(0, CHUNK)], rows_v.at[b], gsem.at[b]
        ).wait()

    def issue_store(j, b):
        pltpu.async_copy(
            rows_v.at[b], out_hbm.at[pl.ds(base + j * CHUNK, CHUNK)], ssem.at[b]
        )

    def wait_store(b):
        pltpu.make_async_copy(
            rows_v.at[b], out_hbm.at[pl.ds(base, CHUNK)], ssem.at[b]
        ).wait()

    for n in range(DEPTH):
        issue_gather(n, n % NBUF)

    for j in range(NBUF):
        wait_gather(j % NBUF)
        issue_store(j, j % NBUF)
        n = j + DEPTH
        if n < NCHUNK:
            if n >= NBUF:
                wait_store(n % NBUF)
            issue_gather(n, n % NBUF)

    def lap(g, carry):
        for b in range(NBUF):
            j = g * NBUF + b
            wait_gather(b)
            issue_store(j, b)
            n = j + DEPTH
            wait_store((b + DEPTH) % NBUF)
            issue_gather(n, (b + DEPTH) % NBUF)
        return carry

    MID_END = ((NCHUNK - DEPTH) // NBUF) * NBUF
    lax.fori_loop(1, MID_END // NBUF, lap, 0)

    for j in range(MID_END, NCHUNK):
        wait_gather(j % NBUF)
        issue_store(j, j % NBUF)
        n = j + DEPTH
        if n < NCHUNK:
            wait_store(n % NBUF)
            issue_gather(n, n % NBUF)

    for b in range(NBUF):
        wait_store(b)


def kernel(indices, table):
    idx3d = indices.astype(jnp.int32).reshape(NUM_WORKERS, NCHUNK, CHUNK)
    out = _gather_kernel(idx3d, table)
    return out.reshape(BATCH, HIST, DIM)


# speedup vs baseline: 1.2854x; 1.2283x over previous
import functools

import jax
import jax.numpy as jnp
from jax import lax
from jax.experimental import pallas as pl
from jax.experimental.pallas import tpu as pltpu
from jax.experimental.pallas import tpu_sc as plsc

VOCAB = 1000000
DIM = 32
BATCH = 4096
HIST = 50

NUM_CORES = 2
NUM_SUBCORES = 16
NUM_WORKERS = NUM_CORES * NUM_SUBCORES

BBLOCK = BATCH // NUM_WORKERS
NCHUNK = HIST
NBUF = 8
DEPTH = 6

_mesh = plsc.VectorSubcoreMesh(core_axis_name="c", subcore_axis_name="s")


@functools.partial(
    pl.kernel,
    mesh=_mesh,
    out_type=jax.ShapeDtypeStruct((BATCH, HIST, DIM), jnp.float32),
    scratch_types=[
        pltpu.VMEM((NCHUNK, BBLOCK), jnp.int32),
        pltpu.VMEM((NBUF, BBLOCK, DIM), jnp.float32),
        pltpu.SemaphoreType.DMA((NBUF,)),
        pltpu.SemaphoreType.DMA((NBUF,)),
    ],
    compiler_params=pltpu.CompilerParams(use_tc_tiling_on_sc=False),
)
def _gather_kernel(idx_hbm, table_hbm, out_hbm, idx_v, rows_v, gsem, ssem):
    wid = lax.axis_index("s") * NUM_CORES + lax.axis_index("c")
    b0 = wid * BBLOCK
    pltpu.sync_copy(idx_hbm.at[:, pl.ds(b0, BBLOCK)], idx_v)

    def issue_gather(n, b):
        pltpu.async_copy(table_hbm.at[idx_v.at[n]], rows_v.at[b], gsem.at[b])

    def wait_gather(b):
        pltpu.make_async_copy(
            table_hbm.at[pl.ds(0, BBLOCK)], rows_v.at[b], gsem.at[b]
        ).wait()

    def issue_store(j, b):
        pltpu.async_copy(
            rows_v.at[b], out_hbm.at[pl.ds(b0, BBLOCK), j], ssem.at[b]
        )

    def wait_store(b):
        pltpu.make_async_copy(
            rows_v.at[b], out_hbm.at[pl.ds(b0, BBLOCK), 0], ssem.at[b]
        ).wait()

    for n in range(DEPTH):
        issue_gather(n, n % NBUF)

    for j in range(NBUF):
        wait_gather(j % NBUF)
        issue_store(j, j % NBUF)
        n = j + DEPTH
        if n < NCHUNK:
            if n >= NBUF:
                wait_store(n % NBUF)
            issue_gather(n, n % NBUF)

    def lap(g, carry):
        for b in range(NBUF):
            j = g * NBUF + b
            wait_gather(b)
            issue_store(j, b)
            wait_store((b + DEPTH) % NBUF)
            issue_gather(j + DEPTH, (b + DEPTH) % NBUF)
        return carry

    MID_END = ((NCHUNK - DEPTH) // NBUF) * NBUF
    lax.fori_loop(1, MID_END // NBUF, lap, 0)

    for j in range(MID_END, NCHUNK):
        wait_gather(j % NBUF)
        issue_store(j, j % NBUF)
        n = j + DEPTH
        if n < NCHUNK:
            wait_store(n % NBUF)
            issue_gather(n, n % NBUF)

    for b in range(NBUF):
        wait_store(b)


def kernel(indices, table):
    idx_t = indices.astype(jnp.int32).T
    return _gather_kernel(idx_t, table)
